# baseline (speedup 1.0000x reference)
import dataclasses
import functools

import jax
import jax.numpy as jnp
from jax import lax
from jax.experimental import pallas as pl
from jax.experimental.pallas import tpu as pltpu
from jax.experimental.pallas import tpu_sc as plsc

N = 10000
E = 320000
D = 128
HID = 16
C = 40
CP = 48

NC = 2
NS = 16
NW = NC * NS
PER_W = E // NW
B = 80
CHUNKS = PER_W // B
LANES = 16

f32 = jnp.float32

_BCAST_DNUMS = lax.GatherDimensionNumbers(
    offset_dims=(), collapsed_slice_dims=(0,), start_index_map=(0,))


def _pre_body(x_ref, w_ref, as_ref, ad_ref,
              h_ref, asrc_ref, adst_ref, wself_ref):
    h = jnp.dot(x_ref[...], w_ref[...], preferred_element_type=f32)
    h_ref[...] = h
    asrc = jnp.dot(h, as_ref[...])
    adst = jnp.dot(h, ad_ref[...])
    asrc_ref[...] = asrc
    adst_ref[...] = adst
    s = asrc + adst
    wself_ref[...] = jnp.exp(jnp.where(s >= 0.0, s, 0.2 * s))


def _mid_body(nump_ref, denp_ref, wself_ref, h_ref, b_ref, gamma_ref,
              beta_ref, w2_ref, as2_ref, ad2_ref,
              h2p_ref, asrc2_ref, adst2_ref, wself2_ref):
    wself = wself_ref[...]
    num = nump_ref[0] + nump_ref[1] + wself * h_ref[...]
    den = jnp.sum(denp_ref[...], axis=0)[:, None] + wself + 1e-16
    out1 = num / den + b_ref[...]
    mean = jnp.mean(out1, axis=0, keepdims=True)
    var = jnp.mean(jnp.square(out1 - mean), axis=0, keepdims=True)
    xn = (out1 - mean) / jnp.sqrt(var + 1e-5) * gamma_ref[...] + beta_ref[...]
    act = jnp.where(xn > 0.0, xn, jnp.exp(xn) - 1.0)
    h2 = jnp.dot(act, w2_ref[...], preferred_element_type=f32)
    h2p_ref[...] = jnp.pad(h2, ((0, 0), (0, CP - C)))
    asrc2 = jnp.dot(h2, as2_ref[...])
    adst2 = jnp.dot(h2, ad2_ref[...])
    asrc2_ref[...] = asrc2
    adst2_ref[...] = adst2
    s = asrc2 + adst2
    wself2_ref[...] = jnp.exp(jnp.where(s >= 0.0, s, 0.2 * s))


def _fin_body(nump_ref, denp_ref, wself_ref, h2p_ref, b_ref, out_ref):
    wself = wself_ref[...]
    num = nump_ref[0] + nump_ref[1] + wself * h2p_ref[...]
    den = jnp.sum(denp_ref[...], axis=0)[:, None] + wself + 1e-16
    out_ref[...] = num[:, :C] / den + b_ref[...]


_pre = pl.pallas_call(
    _pre_body,
    out_shape=(
        jax.ShapeDtypeStruct((N, HID), f32),
        jax.ShapeDtypeStruct((N, 1), f32),
        jax.ShapeDtypeStruct((N, 1), f32),
        jax.ShapeDtypeStruct((N, 1), f32),
    ),
)

_mid = pl.pallas_call(
    _mid_body,
    out_shape=(
        jax.ShapeDtypeStruct((N, CP), f32),
        jax.ShapeDtypeStruct((N, 1), f32),
        jax.ShapeDtypeStruct((N, 1), f32),
        jax.ShapeDtypeStruct((N, 1), f32),
    ),
)

_fin = pl.pallas_call(
    _fin_body,
    out_shape=jax.ShapeDtypeStruct((N, C), f32),
)


def _make_edge_kernel(dw, stage_h=True):
    mesh = plsc.VectorSubcoreMesh(core_axis_name="c", subcore_axis_name="s")
    cp = pltpu.CompilerParams()
    if "needs_layout_passes" in pltpu.CompilerParams.__dataclass_fields__:
        cp = dataclasses.replace(
            cp, needs_layout_passes=False, use_tc_tiling_on_sc=False)

    @functools.partial(
        pl.kernel,
        compiler_params=cp,
        out_type=(
            jax.ShapeDtypeStruct((NC, N, dw), f32),
            jax.ShapeDtypeStruct((NW, N), f32),
        ),
        mesh=mesh,
        scratch_types=[
            pltpu.VMEM((N,), f32),
            pltpu.VMEM((N,), f32),
            pltpu.VMEM((N,), f32),
            pltpu.VMEM((4, B), jnp.int32),
            pltpu.VMEM((4, B), jnp.int32),
            pltpu.VMEM((2, B, dw), f32),
            pltpu.VMEM((2, B, dw), f32),
            pltpu.VMEM_SHARED((N, dw), f32),
            pltpu.VMEM_SHARED(
                (N, dw) if stage_h else (LANES, dw), f32),
            pltpu.SemaphoreType.DMA((4,)),
            pltpu.SemaphoreType.DMA((2,)),
            pltpu.SemaphoreType.DMA((2,)),
            pltpu.SemaphoreType.DMA((3,)),
        ],
    )
    def edge_kernel(src_hbm, dst_hbm, h_hbm, asrc_hbm, adst_hbm, zeros_hbm,
                    nump_hbm, denp_hbm,
                    asrc_v, adst_v, den_v, sidx_v, didx_v, rows_v,
                    wrows_v, num_sh, h_sh, isem, gsem, ssem, psem):
        cid = lax.axis_index("c")
        sid = lax.axis_index("s")
        wid = cid * NS + sid

        def start_idx(ci, r):
            pltpu.async_copy(src_hbm.at[wid].at[ci], sidx_v.at[r], isem.at[r])
            pltpu.async_copy(dst_hbm.at[wid].at[ci], didx_v.at[r], isem.at[r])

        def wait_idx(ci, r):
            pltpu.make_async_copy(
                src_hbm.at[wid].at[ci], sidx_v.at[r], isem.at[r]).wait()
            pltpu.make_async_copy(
                dst_hbm.at[wid].at[ci], didx_v.at[r], isem.at[r]).wait()

        h_tab = h_sh if stage_h else h_hbm

        def start_gather(ci_r, buf):
            pltpu.async_copy(
                h_tab.at[sidx_v.at[ci_r]], rows_v.at[buf], gsem.at[buf])

        def wait_gather(ci_r, buf):
            pltpu.make_async_copy(
                h_tab.at[sidx_v.at[ci_r]], rows_v.at[buf], gsem.at[buf]).wait()

        def start_scatter(r, buf):
            pltpu.async_copy(
                wrows_v.at[buf], num_sh.at[didx_v.at[r]], ssem.at[buf],
                add=True)

        def wait_scatter(r, buf):
            pltpu.make_async_copy(
                wrows_v.at[buf], num_sh.at[didx_v.at[r]], ssem.at[buf]).wait()

        def compute(r, buf):
            for g in range(B // LANES):
                sl = pl.ds(g * LANES, LANES)
                s16 = sidx_v[r, sl]
                d16 = didx_v[r, sl]
                e = (plsc.load_gather(asrc_v, [s16])
                     + plsc.load_gather(adst_v, [d16]))
                e = jnp.where(e >= 0.0, e, 0.2 * e)
                w = jnp.exp(e)
                plsc.addupdate_scatter(den_v, [d16], w)
                wbs = [
                    lax.gather(
                        w, jnp.full((LANES, 1), j, jnp.int32),
                        _BCAST_DNUMS, (1,),
                        mode=lax.GatherScatterMode.PROMISE_IN_BOUNDS)
                    for j in range(LANES)
                ]
                for j in range(LANES):
                    row = g * LANES + j
                    for cc in range(dw // LANES):
                        csl = pl.ds(cc * LANES, LANES)
                        wrows_v[buf, row, csl] = rows_v[buf, row, csl] * wbs[j]

        start_idx(0, 0)
        start_idx(1, 1)
        as_cp = pltpu.async_copy(asrc_hbm, asrc_v, psem.at[0])
        ad_cp = pltpu.async_copy(adst_hbm, adst_v, psem.at[1])
        if stage_h:
            stripe = N // NS
            h_cp = pltpu.async_copy(
                h_hbm.at[pl.ds(sid * stripe, stripe)],
                h_sh.at[pl.ds(sid * stripe, stripe)], psem.at[2])

        @pl.when(sid == 0)
        def _zero_num():
            pltpu.sync_copy(zeros_hbm, num_sh)

        @pl.loop(0, N, step=LANES, unroll=4)
        def _zero_den(i):
            den_v[pl.ds(i, LANES)] = jnp.zeros((LANES,), f32)

        if stage_h:
            h_cp.wait()
        plsc.subcore_barrier()
        wait_idx(0, 0)
        start_gather(0, 0)
        as_cp.wait()
        ad_cp.wait()

        def step(ci, r, b):
            @pl.when(ci >= 2)
            def _():
                wait_scatter((r + 2) % 4, b)

            @pl.when(ci + 2 < CHUNKS)
            def _():
                start_idx(ci + 2, (r + 2) % 4)

            @pl.when(ci + 1 < CHUNKS)
            def _():
                wait_idx(ci + 1, (r + 1) % 4)
                start_gather((r + 1) % 4, 1 - b)

            wait_gather(r, b)
            compute(r, b)
            start_scatter(r, b)

        @pl.loop(0, CHUNKS // 4)
        def _pipe(i):
            ci = i * 4
            step(ci, 0, 0)
            step(ci + 1, 1, 1)
            step(ci + 2, 2, 0)
            step(ci + 3, 3, 1)

        for k in range((CHUNKS // 4) * 4, CHUNKS):
            step(jnp.int32(k), k % 4, k % 2)
        wait_scatter((CHUNKS - 2) % 4, (CHUNKS - 2) % 2)
        wait_scatter((CHUNKS - 1) % 4, (CHUNKS - 1) % 2)

        plsc.subcore_barrier()
        pltpu.sync_copy(den_v, denp_hbm.at[wid])

        @pl.when(sid == 0)
        def _write_num():
            pltpu.sync_copy(num_sh, nump_hbm.at[cid])

    return edge_kernel


_edge1 = _make_edge_kernel(HID, stage_h=True)
_edge2 = _make_edge_kernel(CP, stage_h=False)


@jax.jit
def kernel(x, edge_index, W1, a_s1, a_d1, b1, gamma, beta, W2, a_s2, a_d2, b2):
    src = edge_index[0]
    dst = edge_index[1]

    src = src.reshape(NW, CHUNKS, B)
    dst = dst.reshape(NW, CHUNKS, B)

    h1, asrc1, adst1, wself1 = _pre(x, W1, a_s1[:, None], a_d1[:, None])
    nump1, denp1 = _edge1(
        src, dst, h1,
        asrc1.reshape(N), adst1.reshape(N), jnp.zeros((N, HID), f32))
    h2p, asrc2, adst2, wself2 = _mid(
        nump1, denp1, wself1, h1, b1[None, :], gamma[None, :], beta[None, :],
        W2, a_s2[:, None], a_d2[:, None])
    nump2, denp2 = _edge2(
        src, dst, h2p,
        asrc2.reshape(N), adst2.reshape(N), jnp.zeros((N, CP), f32))
    out = _fin(nump2, denp2, wself2, h2p, b2[None, :])
    return out


# speedup vs baseline: 1.1620x; 1.1620x over previous
import dataclasses
import functools

import jax
import jax.numpy as jnp
from jax import lax
from jax.experimental import pallas as pl
from jax.experimental.pallas import tpu as pltpu
from jax.experimental.pallas import tpu_sc as plsc

N = 10000
E = 320000
D = 128
HID = 16
C = 40
CP = 48

NC = 2
NS = 16
NW = NC * NS
PER_W = E // NW
B = 80
CHUNKS = PER_W // B
LANES = 16

f32 = jnp.float32

_BCAST_DNUMS = lax.GatherDimensionNumbers(
    offset_dims=(), collapsed_slice_dims=(0,), start_index_map=(0,))


def _pre_body(x_ref, w_ref, as_ref, ad_ref,
              h_ref, asrc_ref, adst_ref, wself_ref):
    h = jnp.dot(x_ref[...], w_ref[...], preferred_element_type=f32)
    h_ref[...] = h
    asrc = jnp.dot(h, as_ref[...])
    adst = jnp.dot(h, ad_ref[...])
    asrc_ref[...] = asrc
    adst_ref[...] = adst
    s = asrc + adst
    wself_ref[...] = jnp.exp(jnp.where(s >= 0.0, s, 0.2 * s))


def _mid_body(nump_ref, denp_ref, wself_ref, h_ref, b_ref, gamma_ref,
              beta_ref, w2_ref, as2_ref, ad2_ref,
              act_ref, asrc2_ref, adst2_ref, wself2_ref):
    wself = wself_ref[...]
    num = nump_ref[0] + nump_ref[1] + wself * h_ref[...]
    den = jnp.sum(denp_ref[...], axis=0)[:, None] + wself + 1e-16
    out1 = num / den + b_ref[...]
    mean = jnp.mean(out1, axis=0, keepdims=True)
    var = jnp.mean(jnp.square(out1 - mean), axis=0, keepdims=True)
    xn = (out1 - mean) / jnp.sqrt(var + 1e-5) * gamma_ref[...] + beta_ref[...]
    act = jnp.where(xn > 0.0, xn, jnp.exp(xn) - 1.0)
    act_ref[...] = act
    asrc2 = jnp.dot(act, jnp.dot(w2_ref[...], as2_ref[...]))
    adst2 = jnp.dot(act, jnp.dot(w2_ref[...], ad2_ref[...]))
    asrc2_ref[...] = asrc2
    adst2_ref[...] = adst2
    s = asrc2 + adst2
    wself2_ref[...] = jnp.exp(jnp.where(s >= 0.0, s, 0.2 * s))


def _fin_body(nump_ref, denp_ref, wself_ref, act_ref, w2_ref, b_ref, out_ref):
    wself = wself_ref[...]
    num = nump_ref[0] + nump_ref[1] + wself * act_ref[...]
    den = jnp.sum(denp_ref[...], axis=0)[:, None] + wself + 1e-16
    h2num = jnp.dot(num, w2_ref[...], preferred_element_type=f32)
    out_ref[...] = h2num / den + b_ref[...]


_pre = pl.pallas_call(
    _pre_body,
    out_shape=(
        jax.ShapeDtypeStruct((N, HID), f32),
        jax.ShapeDtypeStruct((N, 1), f32),
        jax.ShapeDtypeStruct((N, 1), f32),
        jax.ShapeDtypeStruct((N, 1), f32),
    ),
)

_mid = pl.pallas_call(
    _mid_body,
    out_shape=(
        jax.ShapeDtypeStruct((N, HID), f32),
        jax.ShapeDtypeStruct((N, 1), f32),
        jax.ShapeDtypeStruct((N, 1), f32),
        jax.ShapeDtypeStruct((N, 1), f32),
    ),
)

_fin = pl.pallas_call(
    _fin_body,
    out_shape=jax.ShapeDtypeStruct((N, C), f32),
)


def _make_edge_kernel(dw, stage_h=True):
    mesh = plsc.VectorSubcoreMesh(core_axis_name="c", subcore_axis_name="s")
    cp = pltpu.CompilerParams()
    if "needs_layout_passes" in pltpu.CompilerParams.__dataclass_fields__:
        cp = dataclasses.replace(
            cp, needs_layout_passes=False, use_tc_tiling_on_sc=False)

    @functools.partial(
        pl.kernel,
        compiler_params=cp,
        out_type=(
            jax.ShapeDtypeStruct((NC, N, dw), f32),
            jax.ShapeDtypeStruct((NW, N), f32),
        ),
        mesh=mesh,
        scratch_types=[
            pltpu.VMEM((N,), f32),
            pltpu.VMEM((N,), f32),
            pltpu.VMEM((N,), f32),
            pltpu.VMEM((4, B), jnp.int32),
            pltpu.VMEM((4, B), jnp.int32),
            pltpu.VMEM((2, B, dw), f32),
            pltpu.VMEM((2, B, dw), f32),
            pltpu.VMEM_SHARED((N, dw), f32),
            pltpu.VMEM_SHARED(
                (N, dw) if stage_h else (LANES, dw), f32),
            pltpu.SemaphoreType.DMA((4,)),
            pltpu.SemaphoreType.DMA((2,)),
            pltpu.SemaphoreType.DMA((2,)),
            pltpu.SemaphoreType.DMA((3,)),
        ],
    )
    def edge_kernel(src_hbm, dst_hbm, h_hbm, asrc_hbm, adst_hbm, zeros_hbm,
                    nump_hbm, denp_hbm,
                    asrc_v, adst_v, den_v, sidx_v, didx_v, rows_v,
                    wrows_v, num_sh, h_sh, isem, gsem, ssem, psem):
        cid = lax.axis_index("c")
        sid = lax.axis_index("s")
        wid = cid * NS + sid

        def start_idx(ci, r):
            pltpu.async_copy(src_hbm.at[wid].at[ci], sidx_v.at[r], isem.at[r])
            pltpu.async_copy(dst_hbm.at[wid].at[ci], didx_v.at[r], isem.at[r])

        def wait_idx(ci, r):
            pltpu.make_async_copy(
                src_hbm.at[wid].at[ci], sidx_v.at[r], isem.at[r]).wait()
            pltpu.make_async_copy(
                dst_hbm.at[wid].at[ci], didx_v.at[r], isem.at[r]).wait()

        h_tab = h_sh if stage_h else h_hbm

        def start_gather(ci_r, buf):
            pltpu.async_copy(
                h_tab.at[sidx_v.at[ci_r]], rows_v.at[buf], gsem.at[buf])

        def wait_gather(ci_r, buf):
            pltpu.make_async_copy(
                h_tab.at[sidx_v.at[ci_r]], rows_v.at[buf], gsem.at[buf]).wait()

        def start_scatter(r, buf):
            pltpu.async_copy(
                wrows_v.at[buf], num_sh.at[didx_v.at[r]], ssem.at[buf],
                add=True)

        def wait_scatter(r, buf):
            pltpu.make_async_copy(
                wrows_v.at[buf], num_sh.at[didx_v.at[r]], ssem.at[buf]).wait()

        def compute(r, buf):
            for g in range(B // LANES):
                sl = pl.ds(g * LANES, LANES)
                s16 = sidx_v[r, sl]
                d16 = didx_v[r, sl]
                e = (plsc.load_gather(asrc_v, [s16])
                     + plsc.load_gather(adst_v, [d16]))
                e = jnp.where(e >= 0.0, e, 0.2 * e)
                w = jnp.exp(e)
                plsc.addupdate_scatter(den_v, [d16], w)
                wbs = [
                    lax.gather(
                        w, jnp.full((LANES, 1), j, jnp.int32),
                        _BCAST_DNUMS, (1,),
                        mode=lax.GatherScatterMode.PROMISE_IN_BOUNDS)
                    for j in range(LANES)
                ]
                for j in range(LANES):
                    row = g * LANES + j
                    for cc in range(dw // LANES):
                        csl = pl.ds(cc * LANES, LANES)
                        wrows_v[buf, row, csl] = rows_v[buf, row, csl] * wbs[j]

        start_idx(0, 0)
        start_idx(1, 1)
        as_cp = pltpu.async_copy(asrc_hbm, asrc_v, psem.at[0])
        ad_cp = pltpu.async_copy(adst_hbm, adst_v, psem.at[1])
        if stage_h:
            stripe = N // NS
            h_cp = pltpu.async_copy(
                h_hbm.at[pl.ds(sid * stripe, stripe)],
                h_sh.at[pl.ds(sid * stripe, stripe)], psem.at[2])

        @pl.when(sid == 0)
        def _zero_num():
            pltpu.sync_copy(zeros_hbm, num_sh)

        @pl.loop(0, N, step=LANES, unroll=4)
        def _zero_den(i):
            den_v[pl.ds(i, LANES)] = jnp.zeros((LANES,), f32)

        if stage_h:
            h_cp.wait()
        plsc.subcore_barrier()
        wait_idx(0, 0)
        start_gather(0, 0)
        as_cp.wait()
        ad_cp.wait()

        def step(ci, r, b):
            @pl.when(ci >= 2)
            def _():
                wait_scatter((r + 2) % 4, b)

            @pl.when(ci + 2 < CHUNKS)
            def _():
                start_idx(ci + 2, (r + 2) % 4)

            @pl.when(ci + 1 < CHUNKS)
            def _():
                wait_idx(ci + 1, (r + 1) % 4)
                start_gather((r + 1) % 4, 1 - b)

            wait_gather(r, b)
            compute(r, b)
            start_scatter(r, b)

        @pl.loop(0, CHUNKS // 4)
        def _pipe(i):
            ci = i * 4
            step(ci, 0, 0)
            step(ci + 1, 1, 1)
            step(ci + 2, 2, 0)
            step(ci + 3, 3, 1)

        for k in range((CHUNKS // 4) * 4, CHUNKS):
            step(jnp.int32(k), k % 4, k % 2)
        wait_scatter((CHUNKS - 2) % 4, (CHUNKS - 2) % 2)
        wait_scatter((CHUNKS - 1) % 4, (CHUNKS - 1) % 2)

        plsc.subcore_barrier()
        pltpu.sync_copy(den_v, denp_hbm.at[wid])

        @pl.when(sid == 0)
        def _write_num():
            pltpu.sync_copy(num_sh, nump_hbm.at[cid])

    return edge_kernel


_edge = _make_edge_kernel(HID, stage_h=True)


@jax.jit
def kernel(x, edge_index, W1, a_s1, a_d1, b1, gamma, beta, W2, a_s2, a_d2, b2):
    src = edge_index[0]
    dst = edge_index[1]

    src = src.reshape(NW, CHUNKS, B)
    dst = dst.reshape(NW, CHUNKS, B)

    zeros = jnp.zeros((N, HID), f32)
    h1, asrc1, adst1, wself1 = _pre(x, W1, a_s1[:, None], a_d1[:, None])
    nump1, denp1 = _edge(
        src, dst, h1, asrc1.reshape(N), adst1.reshape(N), zeros)
    act, asrc2, adst2, wself2 = _mid(
        nump1, denp1, wself1, h1, b1[None, :], gamma[None, :], beta[None, :],
        W2, a_s2[:, None], a_d2[:, None])
    nump2, denp2 = _edge(
        src, dst, act, asrc2.reshape(N), adst2.reshape(N), zeros)
    out = _fin(nump2, denp2, wself2, act, W2, b2[None, :])
    return out
